# baseline (speedup 1.0000x reference)
import functools

import jax
import jax.numpy as jnp
from jax import lax
from jax.experimental import pallas as pl
from jax.experimental.pallas import tpu as pltpu
from jax.experimental.pallas import tpu_sc as plsc

NUM_CORES = 2
NUM_SUBCORES = 16
NUM_WORKERS = NUM_CORES * NUM_SUBCORES

GROUP_ROWS = 8


def _sc_gather(table, idx, *, n_tok, seq, dim):
    rows_per_w = n_tok // NUM_WORKERS
    groups_per_w = rows_per_w // GROUP_ROWS
    n_body = groups_per_w // 2 - 1

    mesh = plsc.VectorSubcoreMesh(core_axis_name="c", subcore_axis_name="s")

    @functools.partial(
        pl.kernel,
        mesh=mesh,
        out_type=jax.ShapeDtypeStruct((n_tok, seq, dim), jnp.float32),
        compiler_params=pltpu.CompilerParams(use_tc_tiling_on_sc=False),
        scratch_types=[
            pltpu.VMEM((rows_per_w, seq), jnp.int32),
            pltpu.VMEM((GROUP_ROWS, seq, dim), jnp.float32),
            pltpu.VMEM((GROUP_ROWS, seq, dim), jnp.float32),
            pltpu.SemaphoreType.DMA,
            pltpu.SemaphoreType.DMA,
            pltpu.SemaphoreType.DMA,
            pltpu.SemaphoreType.DMA,
        ],
    )
    def k(table_hbm, idx_hbm, out_hbm, idx_v, buf_a, buf_b,
          gsem_a, gsem_b, ssem_a, ssem_b):
        wid = lax.axis_index("s") * NUM_CORES + lax.axis_index("c")
        row_base = wid * rows_per_w

        pltpu.sync_copy(idx_hbm.at[pl.ds(row_base, rows_per_w)], idx_v)

        def fire_gathers(buf, gsem, s):
            for r in range(GROUP_ROWS):
                pltpu.async_copy(
                    table_hbm.at[idx_v.at[s * GROUP_ROWS + r]],
                    buf.at[r],
                    gsem)

        def wait_gathers(buf, gsem):
            pltpu.make_async_copy(
                table_hbm.at[pl.ds(0, GROUP_ROWS * seq)],
                buf, gsem).wait()

        def store_slice(s):
            return out_hbm.at[pl.ds(row_base + s * GROUP_ROWS, GROUP_ROWS)]

        def fire_store(buf, ssem, s):
            pltpu.async_copy(buf, store_slice(s), ssem)

        def wait_store(buf, ssem, s):
            pltpu.make_async_copy(buf, store_slice(s), ssem).wait()

        fire_gathers(buf_a, gsem_a, 0)
        fire_gathers(buf_b, gsem_b, 1)

        def body(t, carry):
            s_a = 2 * t
            s_b = s_a + 1
            wait_gathers(buf_a, gsem_a)
            fire_store(buf_a, ssem_a, s_a)
            wait_store(buf_a, ssem_a, s_a)
            fire_gathers(buf_a, gsem_a, s_a + 2)
            wait_gathers(buf_b, gsem_b)
            fire_store(buf_b, ssem_b, s_b)
            wait_store(buf_b, ssem_b, s_b)
            fire_gathers(buf_b, gsem_b, s_b + 2)
            return carry

        lax.fori_loop(0, n_body, body, 0)

        s_last = groups_per_w - 2
        wait_gathers(buf_a, gsem_a)
        fire_store(buf_a, ssem_a, s_last)
        wait_store(buf_a, ssem_a, s_last)
        wait_gathers(buf_b, gsem_b)
        fire_store(buf_b, ssem_b, s_last + 1)
        wait_store(buf_b, ssem_b, s_last + 1)

    return k(table, idx)


def kernel(token_ids, embeddings):
    n_tok, seq = token_ids.shape
    dim = embeddings.shape[1]
    return _sc_gather(embeddings, token_ids.astype(jnp.int32),
                      n_tok=n_tok, seq=seq, dim=dim)
